# Initial kernel scaffold; baseline (speedup 1.0000x reference)
import jax
import jax.numpy as jnp
from jax.experimental import pallas as pl


def kernel(x, W0, b0, gamma0, beta0, W1, b1, gamma1, beta1, W2, b2, edge_index):
    raise NotImplementedError("write your pallas kernel here")


# speedup vs baseline: 16.2861x; 16.2861x over previous
import functools

import jax
import jax.numpy as jnp
from jax import lax
from jax.experimental import pallas as pl
from jax.experimental.pallas import tpu as pltpu
from jax.experimental.pallas import tpu_sc as plsc

N = 10000
NP = 10240
E = 320000
D = 128
H = 256
C = 40
CP = 48
EPS = 1e-5

NC = 2
NS = 16
ROWS_PER_TILE = NP // NS
K = 200

_MESH = dict(core_axis_name="c", subcore_axis_name="s", num_cores=NC,
             num_subcores=NS)


def _deg_body(dst, zeros, ones, dega, degb, dst_v, ones_v, acc, sem):
    del sem
    c = lax.axis_index("c")
    s = lax.axis_index("s")
    pltpu.sync_copy(ones, ones_v)
    pltpu.sync_copy(zeros, acc.at[pl.ds(s * ROWS_PER_TILE, ROWS_PER_TILE)])
    plsc.subcore_barrier()
    per_tile = (E // NC) // NS
    base = c * (E // NC) + s * per_tile

    def chunk(i, carry):
        off = pl.multiple_of(base + i * K, 8)
        pltpu.sync_copy(dst.at[pl.ds(off, K)], dst_v)
        pltpu.sync_copy(ones_v, acc.at[dst_v], add=True)
        return carry

    lax.fori_loop(0, per_tile // K, chunk, 0)
    plsc.subcore_barrier()
    row = pl.ds(s * ROWS_PER_TILE, ROWS_PER_TILE)

    @pl.when(c == 0)
    def _():
        pltpu.sync_copy(acc.at[row], dega.at[row])

    @pl.when(c == 1)
    def _():
        pltpu.sync_copy(acc.at[row], degb.at[row])


def _sc_degree(dst):
    zeros = jnp.zeros((ROWS_PER_TILE, 16), jnp.float32)
    ones = jnp.ones((K, 16), jnp.float32)
    f = pl.kernel(
        _deg_body,
        out_type=[jax.ShapeDtypeStruct((NP, 16), jnp.float32)] * 2,
        mesh=plsc.VectorSubcoreMesh(**_MESH),
        compiler_params=pltpu.CompilerParams(use_tc_tiling_on_sc=False),
        scratch_types=[
            pltpu.VMEM((K,), jnp.int32),
            pltpu.VMEM((K, 16), jnp.float32),
            pltpu.VMEM_SHARED((NP, 16), jnp.float32),
            pltpu.SemaphoreType.DMA,
        ],
    )
    return f(dst, zeros, ones)


def _agg_body(width, split_edges, tab_a, tab_b, src, dst, zeros, out_a,
              out_b, idx_v, rows_v, acc, sem):
    c = lax.axis_index("c")
    s = lax.axis_index("s")
    pltpu.sync_copy(zeros, acc.at[pl.ds(s * ROWS_PER_TILE, ROWS_PER_TILE)])
    plsc.subcore_barrier()
    cnt = (E // NC) if split_edges else E
    per_tile = cnt // NS
    base = (c * cnt if split_edges else 0) + s * per_tile

    def edge_loop(table):
        def chunk(i, carry):
            off = pl.multiple_of(base + i * K, 8)
            pltpu.sync_copy(src.at[pl.ds(off, K)], idx_v)
            pltpu.async_copy(table.at[idx_v], rows_v, sem).wait()
            pltpu.sync_copy(dst.at[pl.ds(off, K)], idx_v)
            pltpu.sync_copy(rows_v, acc.at[idx_v], add=True)
            return carry

        lax.fori_loop(0, per_tile // K, chunk, 0)

    @pl.when(c == 0)
    def _():
        edge_loop(tab_a)

    @pl.when(c == 1)
    def _():
        edge_loop(tab_b)

    plsc.subcore_barrier()
    row = pl.ds(s * ROWS_PER_TILE, ROWS_PER_TILE)

    @pl.when(c == 0)
    def _():
        pltpu.sync_copy(acc.at[row], out_a.at[row])

    @pl.when(c == 1)
    def _():
        pltpu.sync_copy(acc.at[row], out_b.at[row])


def _sc_aggregate(tab_a, tab_b, src, dst, width, split_edges):
    zeros = jnp.zeros((ROWS_PER_TILE, width), jnp.float32)
    f = pl.kernel(
        functools.partial(_agg_body, width, split_edges),
        out_type=[jax.ShapeDtypeStruct((NP, width), jnp.float32)] * 2,
        mesh=plsc.VectorSubcoreMesh(**_MESH),
        compiler_params=pltpu.CompilerParams(use_tc_tiling_on_sc=False),
        scratch_types=[
            pltpu.VMEM((K,), jnp.int32),
            pltpu.VMEM((K, width), jnp.float32),
            pltpu.VMEM_SHARED((NP, width), jnp.float32),
            pltpu.SemaphoreType.DMA,
        ],
    )
    return f(tab_a, tab_b, src, dst, zeros)


def _dinv(dega_ref, degb_ref):
    deg = dega_ref[...][:N, :1] + degb_ref[...][:N, :1] + 1.0
    return lax.rsqrt(deg)


def _tc_u0_body(dega, degb, x, u0):
    u0[...] = x[...] * _dinv(dega, degb)


def _tc_u0(dega, degb, x):
    return pl.pallas_call(
        _tc_u0_body,
        out_shape=jax.ShapeDtypeStruct((N, D), jnp.float32),
    )(dega, degb, x)


def _bn_relu(z, gamma, beta):
    m = jnp.mean(z, axis=0, keepdims=True)
    zc = z - m
    v = jnp.mean(zc * zc, axis=0, keepdims=True)
    return jnp.maximum(zc * lax.rsqrt(v + EPS) * gamma + beta, 0.0)


def _tc_layer1_body(sa, sb, u0, dega, degb, w0, b0, g0, be0, u1a, u1b):
    dinv = _dinv(dega, degb)
    agg = (sa[...][:N] + sb[...][:N] + u0[...]) * dinv
    z = jnp.dot(agg, w0[...], preferred_element_type=jnp.float32) + b0[...]
    h = _bn_relu(z, g0[...], be0[...])
    u1 = h * dinv
    u1a[...] = u1[:, :D]
    u1b[...] = u1[:, D:]


def _tc_layer1(sa, sb, u0, dega, degb, w0, b0, g0, be0):
    return pl.pallas_call(
        _tc_layer1_body,
        out_shape=[jax.ShapeDtypeStruct((N, D), jnp.float32)] * 2,
    )(sa, sb, u0, dega, degb, w0, b0, g0, be0)


def _tc_layer2_body(s1a, s1b, u1a, u1b, dega, degb, w1, b1, g1, be1, w2,
                    u2):
    dinv = _dinv(dega, degb)
    agg = jnp.concatenate(
        [(s1a[...][:N] + u1a[...]) * dinv, (s1b[...][:N] + u1b[...]) * dinv], axis=1)
    z = jnp.dot(agg, w1[...], preferred_element_type=jnp.float32) + b1[...]
    h = _bn_relu(z, g1[...], be1[...])
    t = jnp.dot(h, w2[...], preferred_element_type=jnp.float32)
    u2[...] = t * dinv


def _tc_layer2(s1a, s1b, u1a, u1b, dega, degb, w1, b1, g1, be1, w2):
    return pl.pallas_call(
        _tc_layer2_body,
        out_shape=jax.ShapeDtypeStruct((N, CP), jnp.float32),
    )(s1a, s1b, u1a, u1b, dega, degb, w1, b1, g1, be1, w2)


def _tc_final_body(s2a, s2b, u2, dega, degb, b2, out):
    dinv = _dinv(dega, degb)
    z = ((s2a[...][:N] + s2b[...][:N] + u2[...]) * dinv + b2[...])[:, :C]
    m = jnp.max(z, axis=1, keepdims=True)
    zs = z - m
    lse = jnp.log(jnp.sum(jnp.exp(zs), axis=1, keepdims=True))
    out[...] = zs - lse


def _tc_final(s2a, s2b, u2, dega, degb, b2):
    return pl.pallas_call(
        _tc_final_body,
        out_shape=jax.ShapeDtypeStruct((N, C), jnp.float32),
    )(s2a, s2b, u2, dega, degb, b2)


def kernel(x, W0, b0, gamma0, beta0, W1, b1, gamma1, beta1, W2, b2,
           edge_index):
    src = edge_index[0]
    dst = edge_index[1]
    dega, degb = _sc_degree(dst)

    u0 = _tc_u0(dega, degb, x)
    s0a, s0b = _sc_aggregate(u0, u0, src, dst, D, split_edges=True)
    u1a, u1b = _tc_layer1(s0a, s0b, u0, dega, degb, W0, b0[None, :],
                          gamma0[None, :], beta0[None, :])

    s1a, s1b = _sc_aggregate(u1a, u1b, src, dst, D, split_edges=False)
    w2p = jnp.pad(W2, ((0, 0), (0, CP - C)))
    u2 = _tc_layer2(s1a, s1b, u1a, u1b, dega, degb, W1, b1[None, :],
                    gamma1[None, :], beta1[None, :], w2p)

    s2a, s2b = _sc_aggregate(u2, u2, src, dst, CP, split_edges=True)
    b2p = jnp.pad(b2, (0, CP - C))[None, :]
    return _tc_final(s2a, s2b, u2, dega, degb, b2p)


# speedup vs baseline: 18.6893x; 1.1476x over previous
import functools

import jax
import jax.numpy as jnp
from jax import lax
from jax.experimental import pallas as pl
from jax.experimental.pallas import tpu as pltpu
from jax.experimental.pallas import tpu_sc as plsc

N = 10000
NP = 10240
E = 320000
D = 128
H = 256
C = 40
CP = 48
EPS = 1e-5

NC = 2
NS = 16
ROWS_PER_TILE = NP // NS
K = 80
CH = 125

_MESH = dict(core_axis_name="c", subcore_axis_name="s", num_cores=NC,
             num_subcores=NS)


def _deg_body(dst, zeros, ones, dega, degb, dst_v, ones_v, acc, sem):
    del sem
    c = lax.axis_index("c")
    s = lax.axis_index("s")
    pltpu.sync_copy(ones, ones_v)
    pltpu.sync_copy(zeros, acc.at[pl.ds(s * ROWS_PER_TILE, ROWS_PER_TILE)])
    plsc.subcore_barrier()
    per_tile = (E // NC) // NS
    base = c * (E // NC) + s * per_tile

    def chunk(i, carry):
        off = pl.multiple_of(base + i * K, 8)
        pltpu.sync_copy(dst.at[pl.ds(off, K)], dst_v)
        pltpu.sync_copy(ones_v, acc.at[dst_v], add=True)
        return carry

    lax.fori_loop(0, per_tile // K, chunk, 0)
    plsc.subcore_barrier()
    row = pl.ds(s * ROWS_PER_TILE, ROWS_PER_TILE)

    @pl.when(c == 0)
    def _():
        pltpu.sync_copy(acc.at[row], dega.at[row])

    @pl.when(c == 1)
    def _():
        pltpu.sync_copy(acc.at[row], degb.at[row])


def _sc_degree(dst):
    zeros = jnp.zeros((ROWS_PER_TILE, 16), jnp.float32)
    ones = jnp.ones((K, 16), jnp.float32)
    f = pl.kernel(
        _deg_body,
        out_type=[jax.ShapeDtypeStruct((NP, 16), jnp.float32)] * 2,
        mesh=plsc.VectorSubcoreMesh(**_MESH),
        compiler_params=pltpu.CompilerParams(use_tc_tiling_on_sc=False),
        scratch_types=[
            pltpu.VMEM((K,), jnp.int32),
            pltpu.VMEM((K, 16), jnp.float32),
            pltpu.VMEM_SHARED((NP, 16), jnp.float32),
            pltpu.SemaphoreType.DMA,
        ],
    )
    return f(dst, zeros, ones)


def _agg_body(width, split_edges, tab_a, tab_b, src2, dst2, zeros, out_a,
              out_b, sblk, dblk, r0, r1, acc, gsem0, gsem1, ssem0, ssem1):
    c = lax.axis_index("c")
    s = lax.axis_index("s")
    rows = (r0, r1)
    gsem = (gsem0, gsem1)
    ssem = (ssem0, ssem1)
    pltpu.sync_copy(zeros, acc.at[pl.ds(s * ROWS_PER_TILE, ROWS_PER_TILE)])
    plsc.subcore_barrier()

    def edge_loop(table, chunk_base):
        pltpu.sync_copy(src2.at[pl.ds(chunk_base, CH)], sblk)
        pltpu.sync_copy(dst2.at[pl.ds(chunk_base, CH)], dblk)

        def start_gather(j, p):
            return pltpu.async_copy(table.at[sblk.at[j]], rows[p], gsem[p])

        def start_scatter(j, p):
            pltpu.async_copy(rows[p], acc.at[dblk.at[j]], ssem[p], add=True)

        def drain_scatter(p):
            pltpu.make_async_copy(table.at[pl.ds(0, K)], rows[p],
                                  ssem[p]).wait()

        g0 = start_gather(0, 0)
        g1 = start_gather(1, 1)
        g0.wait()
        start_scatter(0, 0)
        g1.wait()
        start_scatter(1, 1)

        def pair(i2, carry):
            j = 2 + 2 * i2
            for p in (0, 1):
                drain_scatter(p)
                g = start_gather(j + p, p)
                g.wait()
                start_scatter(j + p, p)
            return carry

        lax.fori_loop(0, (CH - 2) // 2, pair, 0)
        if (CH - 2) % 2:
            drain_scatter(0)
            g = start_gather(CH - 1, 0)
            g.wait()
            start_scatter(CH - 1, 0)
        drain_scatter(0)
        drain_scatter(1)

    half_chunks = (E // NC) // K

    @pl.when(c == 0)
    def _():
        edge_loop(tab_a, s * CH)
        if not split_edges:
            edge_loop(tab_a, half_chunks + s * CH)

    @pl.when(c == 1)
    def _():
        edge_loop(tab_b, (half_chunks if split_edges else 0) + s * CH)
        if not split_edges:
            edge_loop(tab_b, half_chunks + s * CH)

    plsc.subcore_barrier()
    row = pl.ds(s * ROWS_PER_TILE, ROWS_PER_TILE)

    @pl.when(c == 0)
    def _():
        pltpu.sync_copy(acc.at[row], out_a.at[row])

    @pl.when(c == 1)
    def _():
        pltpu.sync_copy(acc.at[row], out_b.at[row])


def _sc_aggregate(tab_a, tab_b, src2, dst2, width, split_edges):
    zeros = jnp.zeros((ROWS_PER_TILE, width), jnp.float32)
    f = pl.kernel(
        functools.partial(_agg_body, width, split_edges),
        out_type=[jax.ShapeDtypeStruct((NP, width), jnp.float32)] * 2,
        mesh=plsc.VectorSubcoreMesh(**_MESH),
        compiler_params=pltpu.CompilerParams(use_tc_tiling_on_sc=False),
        scratch_types=[
            pltpu.VMEM((CH, K), jnp.int32),
            pltpu.VMEM((CH, K), jnp.int32),
            pltpu.VMEM((K, width), jnp.float32),
            pltpu.VMEM((K, width), jnp.float32),
            pltpu.VMEM_SHARED((NP, width), jnp.float32),
            pltpu.SemaphoreType.DMA,
            pltpu.SemaphoreType.DMA,
            pltpu.SemaphoreType.DMA,
            pltpu.SemaphoreType.DMA,
        ],
    )
    return f(tab_a, tab_b, src2, dst2, zeros)


def _dinv(dega_ref, degb_ref):
    deg = dega_ref[...][:N, :1] + degb_ref[...][:N, :1] + 1.0
    return lax.rsqrt(deg)


def _tc_u0_body(dega, degb, x, u0):
    u0[...] = x[...] * _dinv(dega, degb)


def _tc_u0(dega, degb, x):
    return pl.pallas_call(
        _tc_u0_body,
        out_shape=jax.ShapeDtypeStruct((N, D), jnp.float32),
    )(dega, degb, x)


def _bn_relu(z, gamma, beta):
    m = jnp.mean(z, axis=0, keepdims=True)
    zc = z - m
    v = jnp.mean(zc * zc, axis=0, keepdims=True)
    return jnp.maximum(zc * lax.rsqrt(v + EPS) * gamma + beta, 0.0)


def _tc_layer1_body(sa, sb, u0, dega, degb, w0, b0, g0, be0, u1a, u1b):
    dinv = _dinv(dega, degb)
    agg = (sa[...][:N] + sb[...][:N] + u0[...]) * dinv
    z = jnp.dot(agg, w0[...], preferred_element_type=jnp.float32) + b0[...]
    h = _bn_relu(z, g0[...], be0[...])
    u1 = h * dinv
    u1a[...] = u1[:, :D]
    u1b[...] = u1[:, D:]


def _tc_layer1(sa, sb, u0, dega, degb, w0, b0, g0, be0):
    return pl.pallas_call(
        _tc_layer1_body,
        out_shape=[jax.ShapeDtypeStruct((N, D), jnp.float32)] * 2,
    )(sa, sb, u0, dega, degb, w0, b0, g0, be0)


def _tc_layer2_body(s1a, s1b, u1a, u1b, dega, degb, w1, b1, g1, be1, w2,
                    u2):
    dinv = _dinv(dega, degb)
    agg = jnp.concatenate(
        [(s1a[...][:N] + u1a[...]) * dinv, (s1b[...][:N] + u1b[...]) * dinv], axis=1)
    z = jnp.dot(agg, w1[...], preferred_element_type=jnp.float32) + b1[...]
    h = _bn_relu(z, g1[...], be1[...])
    t = jnp.dot(h, w2[...], preferred_element_type=jnp.float32)
    u2[...] = t * dinv


def _tc_layer2(s1a, s1b, u1a, u1b, dega, degb, w1, b1, g1, be1, w2):
    return pl.pallas_call(
        _tc_layer2_body,
        out_shape=jax.ShapeDtypeStruct((N, CP), jnp.float32),
    )(s1a, s1b, u1a, u1b, dega, degb, w1, b1, g1, be1, w2)


def _tc_final_body(s2a, s2b, u2, dega, degb, b2, out):
    dinv = _dinv(dega, degb)
    z = ((s2a[...][:N] + s2b[...][:N] + u2[...]) * dinv + b2[...])[:, :C]
    m = jnp.max(z, axis=1, keepdims=True)
    zs = z - m
    lse = jnp.log(jnp.sum(jnp.exp(zs), axis=1, keepdims=True))
    out[...] = zs - lse


def _tc_final(s2a, s2b, u2, dega, degb, b2):
    return pl.pallas_call(
        _tc_final_body,
        out_shape=jax.ShapeDtypeStruct((N, C), jnp.float32),
    )(s2a, s2b, u2, dega, degb, b2)


def kernel(x, W0, b0, gamma0, beta0, W1, b1, gamma1, beta1, W2, b2,
           edge_index):
    src = edge_index[0]
    dst = edge_index[1]
    src2 = src.reshape(E // K, K)
    dst2 = dst.reshape(E // K, K)
    dega, degb = _sc_degree(dst)

    u0 = _tc_u0(dega, degb, x)
    s0a, s0b = _sc_aggregate(u0, u0, src2, dst2, D, split_edges=True)
    u1a, u1b = _tc_layer1(s0a, s0b, u0, dega, degb, W0, b0[None, :],
                          gamma0[None, :], beta0[None, :])

    s1a, s1b = _sc_aggregate(u1a, u1b, src2, dst2, D, split_edges=False)
    w2p = jnp.pad(W2, ((0, 0), (0, CP - C)))
    u2 = _tc_layer2(s1a, s1b, u1a, u1b, dega, degb, W1, b1[None, :],
                    gamma1[None, :], beta1[None, :], w2p)

    s2a, s2b = _sc_aggregate(u2, u2, src2, dst2, CP, split_edges=True)
    b2p = jnp.pad(b2, (0, CP - C))[None, :]
    return _tc_final(s2a, s2b, u2, dega, degb, b2p)


# speedup vs baseline: 27.1570x; 1.4531x over previous
import functools

import jax
import jax.numpy as jnp
from jax import lax
from jax.experimental import pallas as pl
from jax.experimental.pallas import tpu as pltpu
from jax.experimental.pallas import tpu_sc as plsc

N = 10000
NP = 10240
E = 320000
D = 128
H = 256
C = 40
CP = 48
EPS = 1e-5

NC = 2
NS = 16
ROWS_PER_TILE = NP // NS
K = 80
CH = 125
SUB = 25
KDEG = 400

_MESH = dict(core_axis_name="c", subcore_axis_name="s", num_cores=NC,
             num_subcores=NS)


def _deg_body(dst3, zeros, ones, dega, degb, dblk, ones_v, acc, sem):
    c = lax.axis_index("c")
    s = lax.axis_index("s")
    pltpu.sync_copy(ones, ones_v)
    pltpu.sync_copy(zeros, acc.at[pl.ds(s * ROWS_PER_TILE, ROWS_PER_TILE)])
    plsc.subcore_barrier()
    chunks = (E // NC) // KDEG // NS
    base = c * ((E // NC) // KDEG) + s * chunks
    pltpu.sync_copy(dst3.at[pl.ds(base, chunks)], dblk)

    def chunk(j, carry):
        pltpu.async_copy(ones_v, acc.at[dblk.at[j]], sem, add=True).wait()
        return carry

    lax.fori_loop(0, chunks, chunk, 0)
    plsc.subcore_barrier()
    row = pl.ds(s * ROWS_PER_TILE, ROWS_PER_TILE)

    @pl.when(c == 0)
    def _():
        pltpu.sync_copy(acc.at[row], dega.at[row])

    @pl.when(c == 1)
    def _():
        pltpu.sync_copy(acc.at[row], degb.at[row])


def _sc_degree(dst3):
    zeros = jnp.zeros((ROWS_PER_TILE, 16), jnp.float32)
    ones = jnp.ones((KDEG, 16), jnp.float32)
    f = pl.kernel(
        _deg_body,
        out_type=[jax.ShapeDtypeStruct((NP, 16), jnp.float32)] * 2,
        mesh=plsc.VectorSubcoreMesh(**_MESH),
        compiler_params=pltpu.CompilerParams(use_tc_tiling_on_sc=False),
        scratch_types=[
            pltpu.VMEM((25, KDEG), jnp.int32),
            pltpu.VMEM((KDEG, 16), jnp.float32),
            pltpu.VMEM_SHARED((NP, 16), jnp.float32),
            pltpu.SemaphoreType.DMA,
        ],
    )
    return f(dst3, zeros, ones)


def _agg_body(width, split_edges, tab_a, tab_b, src2, dst2, zeros, out_a,
              out_b, sblk, dblk, r0, r1, r2, r3,
              acc, gsem0, gsem1, gsem2, gsem3, ssem0, ssem1, ssem2, ssem3):
    c = lax.axis_index("c")
    s = lax.axis_index("s")
    rows = (r0, r1, r2, r3)
    gsem = (gsem0, gsem1, gsem2, gsem3)
    ssem = (ssem0, ssem1, ssem2, ssem3)
    pltpu.sync_copy(zeros, acc.at[pl.ds(s * ROWS_PER_TILE, ROWS_PER_TILE)])
    plsc.subcore_barrier()

    def edge_loop(table, chunk_base):
        def start_gather(j, p):
            pltpu.async_copy(table.at[sblk.at[j]], rows[p], gsem[p])

        def wait_gather(p):
            pltpu.make_async_copy(table.at[pl.ds(0, K)], rows[p],
                                  gsem[p]).wait()

        def start_scatter(j, p):
            pltpu.async_copy(rows[p], acc.at[dblk.at[j]], ssem[p], add=True)

        def drain_scatter(p):
            pltpu.make_async_copy(table.at[pl.ds(0, K)], rows[p],
                                  ssem[p]).wait()

        def sub_body(sub, carry):
            sb = chunk_base + sub * SUB
            pltpu.sync_copy(src2.at[pl.ds(sb, SUB)], sblk)
            pltpu.sync_copy(dst2.at[pl.ds(sb, SUB)], dblk)
            start_gather(0, 0)
            start_gather(1, 1)
            for j in range(SUB):
                q = (j + 2) % 4
                if j >= 2:
                    drain_scatter(q)
                if j + 2 < SUB:
                    start_gather(j + 2, q)
                wait_gather(j % 4)
                start_scatter(j, j % 4)
            drain_scatter((SUB - 2) % 4)
            drain_scatter((SUB - 1) % 4)
            return carry

        lax.fori_loop(0, CH // SUB, sub_body, 0)

    half_chunks = (E // NC) // K

    @pl.when(c == 0)
    def _():
        edge_loop(tab_a, s * CH)
        if not split_edges:
            edge_loop(tab_a, half_chunks + s * CH)

    @pl.when(c == 1)
    def _():
        edge_loop(tab_b, (half_chunks if split_edges else 0) + s * CH)
        if not split_edges:
            edge_loop(tab_b, half_chunks + s * CH)

    plsc.subcore_barrier()
    row = pl.ds(s * ROWS_PER_TILE, ROWS_PER_TILE)

    @pl.when(c == 0)
    def _():
        pltpu.sync_copy(acc.at[row], out_a.at[row])

    @pl.when(c == 1)
    def _():
        pltpu.sync_copy(acc.at[row], out_b.at[row])


def _sc_aggregate(tab_a, tab_b, src2, dst2, width, split_edges):
    zeros = jnp.zeros((ROWS_PER_TILE, width), jnp.float32)
    f = pl.kernel(
        functools.partial(_agg_body, width, split_edges),
        out_type=[jax.ShapeDtypeStruct((NP, width), jnp.float32)] * 2,
        mesh=plsc.VectorSubcoreMesh(**_MESH),
        compiler_params=pltpu.CompilerParams(use_tc_tiling_on_sc=False),
        scratch_types=[
            pltpu.VMEM((SUB, K), jnp.int32),
            pltpu.VMEM((SUB, K), jnp.int32),
            pltpu.VMEM((K, width), jnp.float32),
            pltpu.VMEM((K, width), jnp.float32),
            pltpu.VMEM((K, width), jnp.float32),
            pltpu.VMEM((K, width), jnp.float32),
            pltpu.VMEM_SHARED((NP, width), jnp.float32),
        ] + [pltpu.SemaphoreType.DMA] * 8,
    )
    return f(tab_a, tab_b, src2, dst2, zeros)


def _dinv(dega_ref, degb_ref):
    deg = dega_ref[...][:N, :1] + degb_ref[...][:N, :1] + 1.0
    return lax.rsqrt(deg)


def _tc_u0_body(dega, degb, x, u0):
    u0[...] = x[...] * _dinv(dega, degb)


def _tc_u0(dega, degb, x):
    return pl.pallas_call(
        _tc_u0_body,
        out_shape=jax.ShapeDtypeStruct((N, D), jnp.float32),
    )(dega, degb, x)


def _bn_relu(z, gamma, beta):
    m = jnp.mean(z, axis=0, keepdims=True)
    zc = z - m
    v = jnp.mean(zc * zc, axis=0, keepdims=True)
    return jnp.maximum(zc * lax.rsqrt(v + EPS) * gamma + beta, 0.0)


def _tc_layer1_body(sa, sb, u0, dega, degb, w0, b0, g0, be0, u1a, u1b):
    dinv = _dinv(dega, degb)
    agg = (sa[...][:N] + sb[...][:N] + u0[...]) * dinv
    z = jnp.dot(agg, w0[...], preferred_element_type=jnp.float32) + b0[...]
    h = _bn_relu(z, g0[...], be0[...])
    u1 = h * dinv
    u1a[...] = u1[:, :D]
    u1b[...] = u1[:, D:]


def _tc_layer1(sa, sb, u0, dega, degb, w0, b0, g0, be0):
    return pl.pallas_call(
        _tc_layer1_body,
        out_shape=[jax.ShapeDtypeStruct((N, D), jnp.float32)] * 2,
    )(sa, sb, u0, dega, degb, w0, b0, g0, be0)


def _tc_layer2_body(s1a, s1b, u1a, u1b, dega, degb, w1, b1, g1, be1, w2,
                    u2):
    dinv = _dinv(dega, degb)
    agg = jnp.concatenate(
        [(s1a[...][:N] + u1a[...]) * dinv, (s1b[...][:N] + u1b[...]) * dinv], axis=1)
    z = jnp.dot(agg, w1[...], preferred_element_type=jnp.float32) + b1[...]
    h = _bn_relu(z, g1[...], be1[...])
    t = jnp.dot(h, w2[...], preferred_element_type=jnp.float32)
    u2[...] = t * dinv


def _tc_layer2(s1a, s1b, u1a, u1b, dega, degb, w1, b1, g1, be1, w2):
    return pl.pallas_call(
        _tc_layer2_body,
        out_shape=jax.ShapeDtypeStruct((N, CP), jnp.float32),
    )(s1a, s1b, u1a, u1b, dega, degb, w1, b1, g1, be1, w2)


def _tc_final_body(s2a, s2b, u2, dega, degb, b2, out):
    dinv = _dinv(dega, degb)
    z = ((s2a[...][:N] + s2b[...][:N] + u2[...]) * dinv + b2[...])[:, :C]
    m = jnp.max(z, axis=1, keepdims=True)
    zs = z - m
    lse = jnp.log(jnp.sum(jnp.exp(zs), axis=1, keepdims=True))
    out[...] = zs - lse


def _tc_final(s2a, s2b, u2, dega, degb, b2):
    return pl.pallas_call(
        _tc_final_body,
        out_shape=jax.ShapeDtypeStruct((N, C), jnp.float32),
    )(s2a, s2b, u2, dega, degb, b2)


def kernel(x, W0, b0, gamma0, beta0, W1, b1, gamma1, beta1, W2, b2,
           edge_index):
    src = edge_index[0]
    dst = edge_index[1]
    src2 = src.reshape(E // K, K)
    dst2 = dst.reshape(E // K, K)
    dega, degb = _sc_degree(dst.reshape(E // KDEG, KDEG))

    u0 = _tc_u0(dega, degb, x)
    s0a, s0b = _sc_aggregate(u0, u0, src2, dst2, D, split_edges=True)
    u1a, u1b = _tc_layer1(s0a, s0b, u0, dega, degb, W0, b0[None, :],
                          gamma0[None, :], beta0[None, :])

    s1a, s1b = _sc_aggregate(u1a, u1b, src2, dst2, D, split_edges=False)
    w2p = jnp.pad(W2, ((0, 0), (0, CP - C)))
    u2 = _tc_layer2(s1a, s1b, u1a, u1b, dega, degb, W1, b1[None, :],
                    gamma1[None, :], beta1[None, :], w2p)

    s2a, s2b = _sc_aggregate(u2, u2, src2, dst2, CP, split_edges=True)
    b2p = jnp.pad(b2, (0, CP - C))[None, :]
    return _tc_final(s2a, s2b, u2, dega, degb, b2p)


# speedup vs baseline: 27.8639x; 1.0260x over previous
import functools

import jax
import jax.numpy as jnp
from jax import lax
from jax.experimental import pallas as pl
from jax.experimental.pallas import tpu as pltpu
from jax.experimental.pallas import tpu_sc as plsc

N = 10000
NP = 10240
E = 320000
D = 128
H = 256
C = 40
CP = 48
EPS = 1e-5

NC = 2
NS = 16
ROWS_PER_TILE = NP // NS
K = 80
CH = 125
SUB = 25
KDEG = 400

_MESH = dict(core_axis_name="c", subcore_axis_name="s", num_cores=NC,
             num_subcores=NS)


def _deg_body(dst3, zeros, ones, dega, degb, dblk, ones_v, acc, sem0, sem1):
    sem = (sem0, sem1)
    c = lax.axis_index("c")
    s = lax.axis_index("s")
    pltpu.sync_copy(ones, ones_v)
    pltpu.sync_copy(zeros, acc.at[pl.ds(s * ROWS_PER_TILE, ROWS_PER_TILE)])
    plsc.subcore_barrier()
    chunks = (E // NC) // KDEG // NS
    base = c * ((E // NC) // KDEG) + s * chunks
    pltpu.sync_copy(dst3.at[pl.ds(base, chunks)], dblk)
    for j in range(chunks):
        if j >= 2:
            pltpu.make_async_copy(ones, ones_v, sem[j % 2]).wait()
        pltpu.async_copy(ones_v, acc.at[dblk.at[j]], sem[j % 2], add=True)
    pltpu.make_async_copy(ones, ones_v, sem[0]).wait()
    pltpu.make_async_copy(ones, ones_v, sem[1]).wait()
    plsc.subcore_barrier()
    row = pl.ds(s * ROWS_PER_TILE, ROWS_PER_TILE)

    @pl.when(c == 0)
    def _():
        pltpu.sync_copy(acc.at[row], dega.at[row])

    @pl.when(c == 1)
    def _():
        pltpu.sync_copy(acc.at[row], degb.at[row])


def _sc_degree(dst3):
    zeros = jnp.zeros((ROWS_PER_TILE, 16), jnp.float32)
    ones = jnp.ones((KDEG, 16), jnp.float32)
    f = pl.kernel(
        _deg_body,
        out_type=[jax.ShapeDtypeStruct((NP, 16), jnp.float32)] * 2,
        mesh=plsc.VectorSubcoreMesh(**_MESH),
        compiler_params=pltpu.CompilerParams(use_tc_tiling_on_sc=False),
        scratch_types=[
            pltpu.VMEM((25, KDEG), jnp.int32),
            pltpu.VMEM((KDEG, 16), jnp.float32),
            pltpu.VMEM_SHARED((NP, 16), jnp.float32),
            pltpu.SemaphoreType.DMA,
            pltpu.SemaphoreType.DMA,
        ],
    )
    return f(dst3, zeros, ones)


def _agg_body(width, split_edges, tab_a, tab_b, src2, dst2, zeros, out_a,
              out_b, sblk, dblk, r0, r1, r2, r3,
              acc, gsem0, gsem1, gsem2, gsem3, ssem0, ssem1, ssem2, ssem3):
    c = lax.axis_index("c")
    s = lax.axis_index("s")
    rows = (r0, r1, r2, r3)
    gsem = (gsem0, gsem1, gsem2, gsem3)
    ssem = (ssem0, ssem1, ssem2, ssem3)
    pltpu.sync_copy(zeros, acc.at[pl.ds(s * ROWS_PER_TILE, ROWS_PER_TILE)])
    plsc.subcore_barrier()

    def edge_loop(table, chunk_base):
        def start_gather(j, p):
            pltpu.async_copy(table.at[sblk.at[j]], rows[p], gsem[p])

        def wait_gather(p):
            pltpu.make_async_copy(table.at[pl.ds(0, K)], rows[p],
                                  gsem[p]).wait()

        def start_scatter(j, p):
            pltpu.async_copy(rows[p], acc.at[dblk.at[j]], ssem[p], add=True)

        def drain_scatter(p):
            pltpu.make_async_copy(table.at[pl.ds(0, K)], rows[p],
                                  ssem[p]).wait()

        def sub_body(sub, carry):
            sb = chunk_base + sub * SUB
            pltpu.sync_copy(src2.at[pl.ds(sb, SUB)], sblk)
            pltpu.sync_copy(dst2.at[pl.ds(sb, SUB)], dblk)
            start_gather(0, 0)
            start_gather(1, 1)
            for j in range(SUB):
                q = (j + 2) % 4
                if j >= 2:
                    drain_scatter(q)
                if j + 2 < SUB:
                    start_gather(j + 2, q)
                wait_gather(j % 4)
                start_scatter(j, j % 4)
            drain_scatter((SUB - 2) % 4)
            drain_scatter((SUB - 1) % 4)
            return carry

        lax.fori_loop(0, CH // SUB, sub_body, 0)

    half_chunks = (E // NC) // K

    @pl.when(c == 0)
    def _():
        edge_loop(tab_a, s * CH)
        if not split_edges:
            edge_loop(tab_a, half_chunks + s * CH)

    @pl.when(c == 1)
    def _():
        edge_loop(tab_b, (half_chunks if split_edges else 0) + s * CH)
        if not split_edges:
            edge_loop(tab_b, half_chunks + s * CH)

    plsc.subcore_barrier()
    row = pl.ds(s * ROWS_PER_TILE, ROWS_PER_TILE)

    @pl.when(c == 0)
    def _():
        pltpu.sync_copy(acc.at[row], out_a.at[row])

    @pl.when(c == 1)
    def _():
        pltpu.sync_copy(acc.at[row], out_b.at[row])


def _sc_aggregate(tab_a, tab_b, src2, dst2, width, split_edges):
    zeros = jnp.zeros((ROWS_PER_TILE, width), jnp.float32)
    f = pl.kernel(
        functools.partial(_agg_body, width, split_edges),
        out_type=[jax.ShapeDtypeStruct((NP, width), jnp.float32)] * 2,
        mesh=plsc.VectorSubcoreMesh(**_MESH),
        compiler_params=pltpu.CompilerParams(use_tc_tiling_on_sc=False),
        scratch_types=[
            pltpu.VMEM((SUB, K), jnp.int32),
            pltpu.VMEM((SUB, K), jnp.int32),
            pltpu.VMEM((K, width), jnp.float32),
            pltpu.VMEM((K, width), jnp.float32),
            pltpu.VMEM((K, width), jnp.float32),
            pltpu.VMEM((K, width), jnp.float32),
            pltpu.VMEM_SHARED((NP, width), jnp.float32),
        ] + [pltpu.SemaphoreType.DMA] * 8,
    )
    return f(tab_a, tab_b, src2, dst2, zeros)


def _agg48_body(tab, src3, dst3, zeros, out_a, out_b, sblk, dblk,
                r0, r1, r2, r3, acc,
                gsem0, gsem1, gsem2, gsem3, ssem0, ssem1, ssem2, ssem3):
    c = lax.axis_index("c")
    s = lax.axis_index("s")
    rows = (r0, r1, r2, r3)
    gsem = (gsem0, gsem1, gsem2, gsem3)
    ssem = (ssem0, ssem1, ssem2, ssem3)
    pltpu.sync_copy(zeros, acc.at[pl.ds(s * ROWS_PER_TILE, ROWS_PER_TILE)])
    plsc.subcore_barrier()
    chunks = (E // NC) // KDEG // NS
    base = c * ((E // NC) // KDEG) + s * chunks
    pltpu.sync_copy(src3.at[pl.ds(base, chunks)], sblk)
    pltpu.sync_copy(dst3.at[pl.ds(base, chunks)], dblk)

    def start_gather(j, p):
        pltpu.async_copy(tab.at[sblk.at[j]], rows[p], gsem[p])

    def wait_gather(p):
        pltpu.make_async_copy(tab.at[pl.ds(0, KDEG)], rows[p],
                              gsem[p]).wait()

    def start_scatter(j, p):
        pltpu.async_copy(rows[p], acc.at[dblk.at[j]], ssem[p], add=True)

    def drain_scatter(p):
        pltpu.make_async_copy(tab.at[pl.ds(0, KDEG)], rows[p],
                              ssem[p]).wait()

    start_gather(0, 0)
    start_gather(1, 1)
    for j in range(chunks):
        q = (j + 2) % 4
        if j >= 2:
            drain_scatter(q)
        if j + 2 < chunks:
            start_gather(j + 2, q)
        wait_gather(j % 4)
        start_scatter(j, j % 4)
    drain_scatter((chunks - 2) % 4)
    drain_scatter((chunks - 1) % 4)
    plsc.subcore_barrier()
    row = pl.ds(s * ROWS_PER_TILE, ROWS_PER_TILE)

    @pl.when(c == 0)
    def _():
        pltpu.sync_copy(acc.at[row], out_a.at[row])

    @pl.when(c == 1)
    def _():
        pltpu.sync_copy(acc.at[row], out_b.at[row])


def _sc_aggregate48(tab, src3, dst3):
    zeros = jnp.zeros((ROWS_PER_TILE, CP), jnp.float32)
    f = pl.kernel(
        _agg48_body,
        out_type=[jax.ShapeDtypeStruct((NP, CP), jnp.float32)] * 2,
        mesh=plsc.VectorSubcoreMesh(**_MESH),
        compiler_params=pltpu.CompilerParams(use_tc_tiling_on_sc=False),
        scratch_types=[
            pltpu.VMEM((25, KDEG), jnp.int32),
            pltpu.VMEM((25, KDEG), jnp.int32),
            pltpu.VMEM((KDEG, CP), jnp.float32),
            pltpu.VMEM((KDEG, CP), jnp.float32),
            pltpu.VMEM((KDEG, CP), jnp.float32),
            pltpu.VMEM((KDEG, CP), jnp.float32),
            pltpu.VMEM_SHARED((NP, CP), jnp.float32),
        ] + [pltpu.SemaphoreType.DMA] * 8,
    )
    return f(tab, src3, dst3, zeros)


def _dinv(dega_ref, degb_ref):
    deg = dega_ref[...][:N, :1] + degb_ref[...][:N, :1] + 1.0
    return lax.rsqrt(deg)


def _tc_u0_body(dega, degb, x, u0):
    u0[...] = x[...] * _dinv(dega, degb)


def _tc_u0(dega, degb, x):
    return pl.pallas_call(
        _tc_u0_body,
        out_shape=jax.ShapeDtypeStruct((N, D), jnp.float32),
    )(dega, degb, x)


def _bn_relu(z, gamma, beta):
    m = jnp.mean(z, axis=0, keepdims=True)
    zc = z - m
    v = jnp.mean(zc * zc, axis=0, keepdims=True)
    return jnp.maximum(zc * lax.rsqrt(v + EPS) * gamma + beta, 0.0)


def _tc_layer1_body(sa, sb, u0, dega, degb, w0, b0, g0, be0, u1a, u1b):
    dinv = _dinv(dega, degb)
    agg = (sa[...][:N] + sb[...][:N] + u0[...]) * dinv
    z = jnp.dot(agg, w0[...], preferred_element_type=jnp.float32) + b0[...]
    h = _bn_relu(z, g0[...], be0[...])
    u1 = h * dinv
    u1a[...] = u1[:, :D]
    u1b[...] = u1[:, D:]


def _tc_layer1(sa, sb, u0, dega, degb, w0, b0, g0, be0):
    return pl.pallas_call(
        _tc_layer1_body,
        out_shape=[jax.ShapeDtypeStruct((N, D), jnp.float32)] * 2,
    )(sa, sb, u0, dega, degb, w0, b0, g0, be0)


def _tc_layer2_body(s1a, s1b, u1a, u1b, dega, degb, w1, b1, g1, be1, w2,
                    u2):
    dinv = _dinv(dega, degb)
    agg = jnp.concatenate(
        [(s1a[...][:N] + u1a[...]) * dinv, (s1b[...][:N] + u1b[...]) * dinv], axis=1)
    z = jnp.dot(agg, w1[...], preferred_element_type=jnp.float32) + b1[...]
    h = _bn_relu(z, g1[...], be1[...])
    t = jnp.dot(h, w2[...], preferred_element_type=jnp.float32)
    u2[...] = t * dinv


def _tc_layer2(s1a, s1b, u1a, u1b, dega, degb, w1, b1, g1, be1, w2):
    return pl.pallas_call(
        _tc_layer2_body,
        out_shape=jax.ShapeDtypeStruct((N, CP), jnp.float32),
    )(s1a, s1b, u1a, u1b, dega, degb, w1, b1, g1, be1, w2)


def _tc_final_body(s2a, s2b, u2, dega, degb, b2, out):
    dinv = _dinv(dega, degb)
    z = ((s2a[...][:N] + s2b[...][:N] + u2[...]) * dinv + b2[...])[:, :C]
    m = jnp.max(z, axis=1, keepdims=True)
    zs = z - m
    lse = jnp.log(jnp.sum(jnp.exp(zs), axis=1, keepdims=True))
    out[...] = zs - lse


def _tc_final(s2a, s2b, u2, dega, degb, b2):
    return pl.pallas_call(
        _tc_final_body,
        out_shape=jax.ShapeDtypeStruct((N, C), jnp.float32),
    )(s2a, s2b, u2, dega, degb, b2)


def kernel(x, W0, b0, gamma0, beta0, W1, b1, gamma1, beta1, W2, b2,
           edge_index):
    src = edge_index[0]
    dst = edge_index[1]
    src2 = src.reshape(E // K, K)
    dst2 = dst.reshape(E // K, K)
    src3 = src.reshape(E // KDEG, KDEG)
    dst3 = dst.reshape(E // KDEG, KDEG)
    dega, degb = _sc_degree(dst3)

    u0 = _tc_u0(dega, degb, x)
    s0a, s0b = _sc_aggregate(u0, u0, src2, dst2, D, split_edges=True)
    u1a, u1b = _tc_layer1(s0a, s0b, u0, dega, degb, W0, b0[None, :],
                          gamma0[None, :], beta0[None, :])

    s1a, s1b = _sc_aggregate(u1a, u1b, src2, dst2, D, split_edges=False)
    w2p = jnp.pad(W2, ((0, 0), (0, CP - C)))
    u2 = _tc_layer2(s1a, s1b, u1a, u1b, dega, degb, W1, b1[None, :],
                    gamma1[None, :], beta1[None, :], w2p)

    s2a, s2b = _sc_aggregate48(u2, src3, dst3)
    b2p = jnp.pad(b2, (0, CP - C))[None, :]
    return _tc_final(s2a, s2b, u2, dega, degb, b2p)
